# Initial kernel scaffold; baseline (speedup 1.0000x reference)
import jax
import jax.numpy as jnp
from jax.experimental import pallas as pl


def kernel(stimuli, targets):
    raise NotImplementedError("write your pallas kernel here")


# speedup vs baseline: 1.6901x; 1.6901x over previous
import jax
import jax.numpy as jnp
from jax.experimental import pallas as pl
from jax.experimental.pallas import tpu as pltpu

N_ROWS = 100000
T_COLS = 256
N_TGT = 128
BLOCK = 4000


def _body(sorted_t_ref, order_ref, starts_ref, stim_ref, o_ref):
    b = pl.program_id(0)
    o_ref[...] = jnp.zeros_like(o_ref)
    lo = starts_ref[b]
    hi = starts_ref[b + 1]

    def add_one(j, carry):
        t = sorted_t_ref[j]
        i = order_ref[j]
        r = t - b * BLOCK
        o_ref[pl.ds(r, 1), :] += stim_ref[pl.ds(i, 1), :]
        return carry

    jax.lax.fori_loop(lo, hi, add_one, 0)


def kernel(stimuli, targets):
    tgt = targets.astype(jnp.int32)
    order = jnp.argsort(tgt).astype(jnp.int32)
    sorted_t = tgt[order]
    edges = (jnp.arange(N_ROWS // BLOCK + 1, dtype=jnp.int32) * BLOCK)
    starts = jnp.searchsorted(sorted_t, edges, side="left").astype(jnp.int32)

    grid_spec = pltpu.PrefetchScalarGridSpec(
        num_scalar_prefetch=3,
        grid=(N_ROWS // BLOCK,),
        in_specs=[
            pl.BlockSpec((N_TGT, T_COLS), lambda b, *_: (0, 0)),
        ],
        out_specs=pl.BlockSpec((BLOCK, T_COLS), lambda b, *_: (b, 0)),
    )
    return pl.pallas_call(
        _body,
        grid_spec=grid_spec,
        out_shape=jax.ShapeDtypeStruct((N_ROWS, T_COLS), jnp.float32),
    )(sorted_t, order, starts, stimuli)
